# Initial kernel scaffold; baseline (speedup 1.0000x reference)
import jax
import jax.numpy as jnp
from jax.experimental import pallas as pl


def kernel(x, cell_id, nodal_values, shape_functions, flag_training, connectivity):
    raise NotImplementedError("write your pallas kernel here")


# speedup vs baseline: 8.9142x; 8.9142x over previous
import functools

import jax
import jax.numpy as jnp
from jax import lax
from jax.experimental import pallas as pl
from jax.experimental.pallas import tpu as pltpu
from jax.experimental.pallas import tpu_sc as plsc

N_CELLS = 128
N_NODES = 130
N_PTS = 16384
NC, NS, L = 2, 16, 16
NW = NC * NS
P_PER_W = N_PTS // NW
VAL_PAD = 160


def _sc_interpolate(cid, sf, vals, conn):
    mesh = plsc.VectorSubcoreMesh(core_axis_name="c", subcore_axis_name="s",
                                  num_cores=NC, num_subcores=NS)

    @functools.partial(
        pl.kernel,
        out_type=jax.ShapeDtypeStruct((2, NW, P_PER_W), jnp.float32),
        mesh=mesh,
        compiler_params=pltpu.CompilerParams(needs_layout_passes=False),
        scratch_types=[
            pltpu.VMEM((P_PER_W,), jnp.int32),
            pltpu.VMEM((P_PER_W,), jnp.float32),
            pltpu.VMEM((P_PER_W,), jnp.float32),
            pltpu.VMEM((P_PER_W,), jnp.float32),
            pltpu.VMEM((2 * VAL_PAD,), jnp.float32),
            pltpu.VMEM((N_CELLS * 3,), jnp.int32),
            pltpu.VMEM((P_PER_W,), jnp.float32),
            pltpu.VMEM((P_PER_W,), jnp.float32),
        ],
    )
    def body(cid_hbm, sf_hbm, vals_hbm, conn_hbm, out_hbm,
             cid_v, sf0_v, sf1_v, sf2_v, vals_v, conn_v, out0_v, out1_v):
        wid = lax.axis_index("s") * NC + lax.axis_index("c")
        pltpu.sync_copy(cid_hbm.at[wid], cid_v)
        pltpu.sync_copy(sf_hbm.at[0, wid], sf0_v)
        pltpu.sync_copy(sf_hbm.at[1, wid], sf1_v)
        pltpu.sync_copy(sf_hbm.at[2, wid], sf2_v)
        pltpu.sync_copy(vals_hbm, vals_v)
        pltpu.sync_copy(conn_hbm, conn_v)

        sf_refs = (sf0_v, sf1_v, sf2_v)
        for i in range(P_PER_W // L):
            sl = pl.ds(i * L, L)
            cid3 = cid_v[sl] * 3
            acc0 = jnp.zeros((L,), jnp.float32)
            acc1 = jnp.zeros((L,), jnp.float32)
            for j in range(3):
                node = plsc.load_gather(conn_v, [cid3 + j]) - 1
                w = sf_refs[j][sl]
                acc0 = acc0 + w * plsc.load_gather(vals_v, [node])
                acc1 = acc1 + w * plsc.load_gather(vals_v, [node + VAL_PAD])
            out0_v[sl] = acc0
            out1_v[sl] = acc1

        pltpu.sync_copy(out0_v, out_hbm.at[0, wid])
        pltpu.sync_copy(out1_v, out_hbm.at[1, wid])

    return body(cid, sf, vals, conn)


@jax.jit
def kernel(x, cell_id, nodal_values, shape_functions, flag_training,
           connectivity):
    del x, flag_training
    cid = cell_id.astype(jnp.int32).reshape(NW, P_PER_W)
    sf = shape_functions.astype(jnp.float32).T.reshape(3, NW, P_PER_W)
    vals = jnp.pad(nodal_values[:, :, 0].astype(jnp.float32),
                   ((0, 0), (0, VAL_PAD - N_NODES))).reshape(2 * VAL_PAD)
    conn = connectivity.astype(jnp.int32).reshape(N_CELLS * 3)
    out = _sc_interpolate(cid, sf, vals, conn)
    return out.reshape(2, N_PTS)


# speedup vs baseline: 9.6942x; 1.0875x over previous
import functools

import jax
import jax.numpy as jnp
from jax import lax
from jax.experimental import pallas as pl
from jax.experimental.pallas import tpu as pltpu
from jax.experimental.pallas import tpu_sc as plsc

N_CELLS = 128
N_NODES = 130
N_PTS = 16384
NC, NS, L = 2, 16, 16
NW = NC * NS
P_PER_W = N_PTS // NW
VAL_PAD = 160


def _sc_interpolate(cid, sf, vals, conn):
    mesh = plsc.VectorSubcoreMesh(core_axis_name="c", subcore_axis_name="s",
                                  num_cores=NC, num_subcores=NS)

    @functools.partial(
        pl.kernel,
        out_type=jax.ShapeDtypeStruct((2, NW, P_PER_W), jnp.float32),
        mesh=mesh,
        compiler_params=pltpu.CompilerParams(needs_layout_passes=False),
        scratch_types=[
            pltpu.VMEM((P_PER_W,), jnp.int32),
            pltpu.VMEM((P_PER_W,), jnp.float32),
            pltpu.VMEM((P_PER_W,), jnp.float32),
            pltpu.VMEM((P_PER_W,), jnp.float32),
            pltpu.VMEM((2 * VAL_PAD,), jnp.float32),
            pltpu.VMEM((N_CELLS * 3,), jnp.int32),
            pltpu.VMEM((P_PER_W,), jnp.float32),
            pltpu.VMEM((P_PER_W,), jnp.float32),
            pltpu.SemaphoreType.DMA,
        ],
    )
    def body(cid_hbm, sf_hbm, vals_hbm, conn_hbm, out_hbm,
             cid_v, sf0_v, sf1_v, sf2_v, vals_v, conn_v, out0_v, out1_v,
             sem):
        wid = lax.axis_index("s") * NC + lax.axis_index("c")
        copies = [
            pltpu.async_copy(cid_hbm.at[wid], cid_v, sem),
            pltpu.async_copy(sf_hbm.at[0, wid], sf0_v, sem),
            pltpu.async_copy(sf_hbm.at[1, wid], sf1_v, sem),
            pltpu.async_copy(sf_hbm.at[2, wid], sf2_v, sem),
            pltpu.async_copy(vals_hbm, vals_v, sem),
            pltpu.async_copy(conn_hbm, conn_v, sem),
        ]
        for c in copies:
            c.wait()

        sf_refs = (sf0_v, sf1_v, sf2_v)
        for i in range(P_PER_W // L):
            sl = pl.ds(i * L, L)
            cid3 = cid_v[sl] * 3
            acc0 = jnp.zeros((L,), jnp.float32)
            acc1 = jnp.zeros((L,), jnp.float32)
            for j in range(3):
                node = plsc.load_gather(conn_v, [cid3 + j]) - 1
                w = sf_refs[j][sl]
                acc0 = acc0 + w * plsc.load_gather(vals_v, [node])
                acc1 = acc1 + w * plsc.load_gather(vals_v, [node + VAL_PAD])
            out0_v[sl] = acc0
            out1_v[sl] = acc1

        pltpu.sync_copy(out0_v, out_hbm.at[0, wid])
        pltpu.sync_copy(out1_v, out_hbm.at[1, wid])

    return body(cid, sf, vals, conn)


@jax.jit
def kernel(x, cell_id, nodal_values, shape_functions, flag_training,
           connectivity):
    del x, flag_training
    cid = cell_id.astype(jnp.int32).reshape(NW, P_PER_W)
    sf = shape_functions.astype(jnp.float32).T.reshape(3, NW, P_PER_W)
    vals = jnp.pad(nodal_values[:, :, 0].astype(jnp.float32),
                   ((0, 0), (0, VAL_PAD - N_NODES))).reshape(2 * VAL_PAD)
    conn = connectivity.astype(jnp.int32).reshape(N_CELLS * 3)
    out = _sc_interpolate(cid, sf, vals, conn)
    return out.reshape(2, N_PTS)


# speedup vs baseline: 9.9951x; 1.0310x over previous
import functools

import jax
import jax.numpy as jnp
from jax import lax
from jax.experimental import pallas as pl
from jax.experimental.pallas import tpu as pltpu
from jax.experimental.pallas import tpu_sc as plsc

N_CELLS = 128
N_NODES = 130
N_PTS = 16384
NC, NS, L = 2, 16, 16
NW = NC * NS
P_PER_W = N_PTS // NW
VAL_PAD = 160


def _sc_interpolate(cid, sf, vals, conn):
    mesh = plsc.VectorSubcoreMesh(core_axis_name="c", subcore_axis_name="s",
                                  num_cores=NC, num_subcores=NS)

    @functools.partial(
        pl.kernel,
        out_type=jax.ShapeDtypeStruct((2, NW, P_PER_W), jnp.float32),
        mesh=mesh,
        compiler_params=pltpu.CompilerParams(needs_layout_passes=False),
        scratch_types=[
            pltpu.VMEM((P_PER_W,), jnp.int32),
            pltpu.VMEM((P_PER_W,), jnp.float32),
            pltpu.VMEM((P_PER_W,), jnp.float32),
            pltpu.VMEM((P_PER_W,), jnp.float32),
            pltpu.VMEM((2 * VAL_PAD,), jnp.float32),
            pltpu.VMEM((N_CELLS * 3,), jnp.int32),
            pltpu.VMEM((P_PER_W,), jnp.float32),
            pltpu.VMEM((P_PER_W,), jnp.float32),
            pltpu.SemaphoreType.DMA,
        ],
    )
    def body(cid_hbm, sf_hbm, vals_hbm, conn_hbm, out_hbm,
             cid_v, sf0_v, sf1_v, sf2_v, vals_v, conn_v, out0_v, out1_v,
             sem):
        wid = lax.axis_index("s") * NC + lax.axis_index("c")
        copies = [
            pltpu.async_copy(cid_hbm.at[wid], cid_v, sem),
            pltpu.async_copy(sf_hbm.at[0, wid], sf0_v, sem),
            pltpu.async_copy(sf_hbm.at[1, wid], sf1_v, sem),
            pltpu.async_copy(sf_hbm.at[2, wid], sf2_v, sem),
            pltpu.async_copy(vals_hbm, vals_v, sem),
            pltpu.async_copy(conn_hbm, conn_v, sem),
        ]
        for c in copies:
            c.wait()

        sf_refs = (sf0_v, sf1_v, sf2_v)

        @plsc.parallel_loop(0, P_PER_W, step=L, unroll=4)
        def _loop(i):
            sl = pl.ds(i, L)
            cid3 = cid_v[sl] * 3
            acc0 = jnp.zeros((L,), jnp.float32)
            acc1 = jnp.zeros((L,), jnp.float32)
            for j in range(3):
                node = plsc.load_gather(conn_v, [cid3 + j]) - 1
                w = sf_refs[j][sl]
                acc0 = acc0 + w * plsc.load_gather(vals_v, [node])
                acc1 = acc1 + w * plsc.load_gather(vals_v, [node + VAL_PAD])
            out0_v[sl] = acc0
            out1_v[sl] = acc1

        pltpu.sync_copy(out0_v, out_hbm.at[0, wid])
        pltpu.sync_copy(out1_v, out_hbm.at[1, wid])

    return body(cid, sf, vals, conn)


@jax.jit
def kernel(x, cell_id, nodal_values, shape_functions, flag_training,
           connectivity):
    del x, flag_training
    cid = cell_id.astype(jnp.int32).reshape(NW, P_PER_W)
    sf = shape_functions.astype(jnp.float32).T.reshape(3, NW, P_PER_W)
    vals = jnp.pad(nodal_values[:, :, 0].astype(jnp.float32),
                   ((0, 0), (0, VAL_PAD - N_NODES))).reshape(2 * VAL_PAD)
    conn = connectivity.astype(jnp.int32).reshape(N_CELLS * 3)
    out = _sc_interpolate(cid, sf, vals, conn)
    return out.reshape(2, N_PTS)
